# baseline (speedup 1.0000x reference)
import functools

import jax
import jax.numpy as jnp
from jax import lax
from jax.experimental import pallas as pl
from jax.experimental.pallas import tpu as pltpu
from jax.experimental.pallas import tpu_sc as plsc

_B = 1024
_S = 200
_D = 64
_ROWS = _B * _S
_NW = 32
_RPW = _ROWS // _NW
_C = 128
_NCHUNK = _RPW // _C
_L = 16


_GDN = lax.GatherDimensionNumbers(
    offset_dims=(), collapsed_slice_dims=(0,), start_index_map=(0,))


def _shuffle(v, p):
    return lax.gather(v, p[:, None], _GDN, slice_sizes=(1,),
                      mode=lax.GatherScatterMode.PROMISE_IN_BOUNDS)


def _lanesum(v, perms):
    for p in perms:
        v = v + _shuffle(v, p)
    return v


def _rsqrt16(a):
    ai = lax.bitcast_convert_type(a, jnp.int32)
    yi = jnp.int32(0x5F3759DF) - lax.shift_right_arithmetic(ai, jnp.int32(1))
    y = lax.bitcast_convert_type(yi, jnp.float32)
    h = a * jnp.float32(0.5)
    for _ in range(3):
        y = y * (jnp.float32(1.5) - h * y * y)
    return y


def _sc_kernel(idx_hbm, table_hbm, gamma_hbm, beta_hbm, pe2_hbm, out_hbm,
               idx_v, emb_v, pe_v, g_v, b_v, sem):
    wid = lax.axis_index("s") * 2 + lax.axis_index("c")
    base = wid * _RPW

    pltpu.sync_copy(pe2_hbm, pe_v)
    pltpu.sync_copy(gamma_hbm, g_v)
    pltpu.sync_copy(beta_hbm, b_v)

    g = [g_v[pl.ds(j * _L, _L)] for j in range(4)]
    b = [b_v[pl.ds(j * _L, _L)] for j in range(4)]
    inv_d = jnp.float32(1.0 / _D)
    lane = lax.iota(jnp.int32, _L)
    perms = [lax.bitwise_xor(lane, jnp.int32(k)) for k in (8, 4, 2, 1)]

    def chunk_body(ci, carry):
        cbase = base + ci * _C
        pbase = lax.rem(ci * _C, jnp.int32(_S))
        pltpu.sync_copy(idx_hbm.at[pl.ds(cbase, _C)], idx_v)
        pltpu.async_copy(table_hbm.at[idx_v], emb_v, sem).wait()

        def row_body(r, rcarry):
            pos = pbase + r
            x = [emb_v[r, pl.ds(j * _L, _L)] + pe_v[pos, pl.ds(j * _L, _L)]
                 for j in range(4)]
            s1 = (x[0] + x[1]) + (x[2] + x[3])
            s2 = ((x[0] * x[0] + x[1] * x[1])
                  + (x[2] * x[2] + x[3] * x[3]))
            m = _lanesum(s1, perms) * inv_d
            ex2 = _lanesum(s2, perms) * inv_d
            var = ex2 - m * m
            r_std = _rsqrt16(var + jnp.float32(1e-5))
            for j in range(4):
                emb_v[r, pl.ds(j * _L, _L)] = (x[j] - m) * r_std * g[j] + b[j]
            return rcarry

        lax.fori_loop(0, _C, row_body, 0, unroll=False)
        pltpu.sync_copy(emb_v, out_hbm.at[pl.ds(cbase, _C)])
        return carry

    lax.fori_loop(0, _NCHUNK, chunk_body, 0, unroll=False)


def kernel(token_ids, table, gamma, beta, pe):
    idx_flat = token_ids.reshape(_ROWS)
    pe2 = jnp.concatenate([pe, pe], axis=0)

    mesh = plsc.VectorSubcoreMesh(core_axis_name="c", subcore_axis_name="s")
    run = functools.partial(
        pl.kernel,
        mesh=mesh,
        compiler_params=pltpu.CompilerParams(use_tc_tiling_on_sc=False),
        out_type=jax.ShapeDtypeStruct((_ROWS, _D), jnp.float32),
        scratch_types=[
            pltpu.VMEM((_C,), jnp.int32),
            pltpu.VMEM((_C, _D), jnp.float32),
            pltpu.VMEM((2 * _S, _D), jnp.float32),
            pltpu.VMEM((_D,), jnp.float32),
            pltpu.VMEM((_D,), jnp.float32),
            pltpu.SemaphoreType.DMA,
        ],
    )(_sc_kernel)
    out = run(idx_flat, table, gamma, beta, pe2)
    return out.reshape(_B, _S, _D)


# speedup vs baseline: 1.0015x; 1.0015x over previous
import functools

import jax
import jax.numpy as jnp
from jax import lax
from jax.experimental import pallas as pl
from jax.experimental.pallas import tpu as pltpu
from jax.experimental.pallas import tpu_sc as plsc

_B = 1024
_S = 200
_D = 64
_ROWS = _B * _S
_NW = 32
_RPW = _ROWS // _NW
_C = 128
_NCHUNK = _RPW // _C
_L = 16


_GDN = lax.GatherDimensionNumbers(
    offset_dims=(), collapsed_slice_dims=(0,), start_index_map=(0,))


def _shuffle(v, p):
    return lax.gather(v, p[:, None], _GDN, slice_sizes=(1,),
                      mode=lax.GatherScatterMode.PROMISE_IN_BOUNDS)


def _lanesum(v, perms):
    for p in perms:
        v = v + _shuffle(v, p)
    return v


def _rsqrt16(a):
    ai = lax.bitcast_convert_type(a, jnp.int32)
    yi = jnp.int32(0x5F3759DF) - lax.shift_right_arithmetic(ai, jnp.int32(1))
    y = lax.bitcast_convert_type(yi, jnp.float32)
    h = a * jnp.float32(0.5)
    for _ in range(3):
        y = y * (jnp.float32(1.5) - h * y * y)
    return y


def _sc_kernel(idx_hbm, table_hbm, gamma_hbm, beta_hbm, pe2_hbm, out_hbm,
               idx_v, emb_v, pe_v, g_v, b_v, sem):
    wid = lax.axis_index("s") * 2 + lax.axis_index("c")
    base = wid * _RPW

    pltpu.sync_copy(pe2_hbm, pe_v)
    pltpu.sync_copy(gamma_hbm, g_v)
    pltpu.sync_copy(beta_hbm, b_v)

    g = [g_v[pl.ds(j * _L, _L)] for j in range(4)]
    b = [b_v[pl.ds(j * _L, _L)] for j in range(4)]
    inv_d = jnp.float32(1.0 / _D)
    lane = lax.iota(jnp.int32, _L)
    perms = [lax.bitwise_xor(lane, jnp.int32(k)) for k in (8, 4, 2, 1)]

    def chunk_body(ci, carry):
        cbase = base + ci * _C
        pbase = lax.rem(ci * _C, jnp.int32(_S))
        pltpu.sync_copy(idx_hbm.at[pl.ds(cbase, _C)], idx_v)
        pltpu.async_copy(table_hbm.at[idx_v], emb_v, sem).wait()

        def row_body(r, rcarry):
            pos = pbase + r
            x = [emb_v[r, pl.ds(j * _L, _L)] + pe_v[pos, pl.ds(j * _L, _L)]
                 for j in range(4)]
            s1 = (x[0] + x[1]) + (x[2] + x[3])
            s2 = ((x[0] * x[0] + x[1] * x[1])
                  + (x[2] * x[2] + x[3] * x[3]))
            m = _lanesum(s1, perms) * inv_d
            ex2 = _lanesum(s2, perms) * inv_d
            var = ex2 - m * m
            r_std = _rsqrt16(var + jnp.float32(1e-5))
            for j in range(4):
                emb_v[r, pl.ds(j * _L, _L)] = (x[j] - m) * r_std * g[j] + b[j]
            return rcarry

        lax.fori_loop(0, _C, row_body, 0, unroll=8)
        pltpu.sync_copy(emb_v, out_hbm.at[pl.ds(cbase, _C)])
        return carry

    lax.fori_loop(0, _NCHUNK, chunk_body, 0, unroll=False)


def kernel(token_ids, table, gamma, beta, pe):
    idx_flat = token_ids.reshape(_ROWS)
    pe2 = jnp.concatenate([pe, pe], axis=0)

    mesh = plsc.VectorSubcoreMesh(core_axis_name="c", subcore_axis_name="s")
    run = functools.partial(
        pl.kernel,
        mesh=mesh,
        compiler_params=pltpu.CompilerParams(use_tc_tiling_on_sc=False),
        out_type=jax.ShapeDtypeStruct((_ROWS, _D), jnp.float32),
        scratch_types=[
            pltpu.VMEM((_C,), jnp.int32),
            pltpu.VMEM((_C, _D), jnp.float32),
            pltpu.VMEM((2 * _S, _D), jnp.float32),
            pltpu.VMEM((_D,), jnp.float32),
            pltpu.VMEM((_D,), jnp.float32),
            pltpu.SemaphoreType.DMA,
        ],
    )(_sc_kernel)
    out = run(idx_flat, table, gamma, beta, pe2)
    return out.reshape(_B, _S, _D)


# speedup vs baseline: 1.0658x; 1.0642x over previous
import functools

import jax
import jax.numpy as jnp
from jax import lax
from jax.experimental import pallas as pl
from jax.experimental.pallas import tpu as pltpu
from jax.experimental.pallas import tpu_sc as plsc

_B = 1024
_S = 200
_D = 64
_NW = 32
_SPW = _B // _NW
_L = 16
_SPLITS = ((0, 128), (128, 72))


_GDN = lax.GatherDimensionNumbers(
    offset_dims=(), collapsed_slice_dims=(0,), start_index_map=(0,))


def _shuffle(v, p):
    return lax.gather(v, p[:, None], _GDN, slice_sizes=(1,),
                      mode=lax.GatherScatterMode.PROMISE_IN_BOUNDS)


def _lanesum(v, perms):
    for p in perms:
        v = v + _shuffle(v, p)
    return v


def _rsqrt16(a):
    ai = lax.bitcast_convert_type(a, jnp.int32)
    yi = jnp.int32(0x5F3759DF) - lax.shift_right_arithmetic(ai, jnp.int32(1))
    y = lax.bitcast_convert_type(yi, jnp.float32)
    h = a * jnp.float32(0.5)
    for _ in range(3):
        y = y * (jnp.float32(1.5) - h * y * y)
    return y


def _sc_kernel(idx_hbm, table_hbm, gamma_hbm, beta_hbm, pe_hbm, out_hbm,
               idx_v, emb_v, pe_v, g_v, b_v, gsem0, gsem1, osem0, osem1):
    wid = lax.axis_index("s") * 2 + lax.axis_index("c")
    sbase = wid * _SPW

    pltpu.sync_copy(pe_hbm, pe_v)
    pltpu.sync_copy(gamma_hbm, g_v)
    pltpu.sync_copy(beta_hbm, b_v)
    pltpu.sync_copy(idx_hbm.at[pl.ds(sbase, _SPW)], idx_v)

    g = [g_v[pl.ds(j * _L, _L)] for j in range(4)]
    b = [b_v[pl.ds(j * _L, _L)] for j in range(4)]
    inv_d = jnp.float32(1.0 / _D)
    lane = lax.iota(jnp.int32, _L)
    perms = [lax.bitwise_xor(lane, jnp.int32(k)) for k in (8, 4, 2, 1)]

    emb0 = emb_v.at[0]
    emb1 = emb_v.at[1]

    def gather_start(ci, emb_b, gsem):
        for o, n in _SPLITS:
            pltpu.make_async_copy(
                table_hbm.at[idx_v.at[ci, pl.ds(o, n)]],
                emb_b.at[pl.ds(o, n)], gsem).start()

    def gather_wait(emb_b, gsem):
        for o, n in _SPLITS:
            pltpu.make_async_copy(
                table_hbm.at[idx_v.at[0, pl.ds(o, n)]],
                emb_b.at[pl.ds(o, n)], gsem).wait()

    def out_start(ci, emb_b, osem):
        pltpu.make_async_copy(emb_b, out_hbm.at[sbase + ci], osem).start()

    def out_wait(emb_b, osem):
        pltpu.make_async_copy(emb_b, out_hbm.at[0], osem).wait()

    def compute(emb_b):
        def row_body(r, rcarry):
            x = [emb_b[r, pl.ds(j * _L, _L)] + pe_v[r, pl.ds(j * _L, _L)]
                 for j in range(4)]
            s1 = (x[0] + x[1]) + (x[2] + x[3])
            s2 = ((x[0] * x[0] + x[1] * x[1])
                  + (x[2] * x[2] + x[3] * x[3]))
            m = _lanesum(s1, perms) * inv_d
            ex2 = _lanesum(s2, perms) * inv_d
            var = ex2 - m * m
            r_std = _rsqrt16(var + jnp.float32(1e-5))
            for j in range(4):
                emb_b[r, pl.ds(j * _L, _L)] = (x[j] - m) * r_std * g[j] + b[j]
            return rcarry

        lax.fori_loop(0, _S, row_body, 0, unroll=8)

    gather_start(0, emb0, gsem0)

    def body(i, carry):
        c0 = 2 * i
        c1 = c0 + 1

        @pl.when(i >= 1)
        def _():
            out_wait(emb1, osem1)

        gather_start(c1, emb1, gsem1)
        gather_wait(emb0, gsem0)
        compute(emb0)
        out_start(c0, emb0, osem0)
        gather_wait(emb1, gsem1)
        compute(emb1)
        out_wait(emb0, osem0)

        @pl.when(i <= _SPW // 2 - 2)
        def _():
            gather_start(c0 + 2, emb0, gsem0)

        out_start(c1, emb1, osem1)
        return carry

    lax.fori_loop(0, _SPW // 2, body, 0)
    out_wait(emb1, osem1)


def kernel(token_ids, table, gamma, beta, pe):
    mesh = plsc.VectorSubcoreMesh(core_axis_name="c", subcore_axis_name="s")
    run = functools.partial(
        pl.kernel,
        mesh=mesh,
        compiler_params=pltpu.CompilerParams(use_tc_tiling_on_sc=False),
        out_type=jax.ShapeDtypeStruct((_B, _S, _D), jnp.float32),
        scratch_types=[
            pltpu.VMEM((_SPW, _S), jnp.int32),
            pltpu.VMEM((2, _S, _D), jnp.float32),
            pltpu.VMEM((_S, _D), jnp.float32),
            pltpu.VMEM((_D,), jnp.float32),
            pltpu.VMEM((_D,), jnp.float32),
            pltpu.SemaphoreType.DMA,
            pltpu.SemaphoreType.DMA,
            pltpu.SemaphoreType.DMA,
            pltpu.SemaphoreType.DMA,
        ],
    )(_sc_kernel)
    return run(token_ids, table, gamma, beta, pe)
